# baseline (speedup 1.0000x reference)
import functools

import jax
import jax.numpy as jnp
from jax import lax
from jax.experimental import pallas as pl
from jax.experimental.pallas import tpu as pltpu
from jax.experimental.pallas import tpu_sc as plsc

_D = 32
_CH = 32
_HP = 256


def _sc_transe(lhs, rel, rhs, ent_embeds, rel_embeds, B):
    D = _D
    info = plsc.get_sparse_core_info()
    nw = info.num_cores * info.num_subcores
    bpw = B // nw
    npass = bpw // _HP
    nchp = _HP // _CH

    mesh = plsc.VectorSubcoreMesh(core_axis_name="c", subcore_axis_name="s")

    @functools.partial(
        pl.kernel,
        mesh=mesh,
        compiler_params=pltpu.CompilerParams(
            needs_layout_passes=False, skip_device_barrier=True),
        out_type=jax.ShapeDtypeStruct((B,), jnp.float32),
        scratch_types=[
            pltpu.VMEM((bpw,), jnp.int32),
            pltpu.VMEM((bpw,), jnp.int32),
            pltpu.VMEM((bpw,), jnp.int32),
            pltpu.VMEM((_HP, D), jnp.float32),
            pltpu.VMEM((_HP, D), jnp.float32),
            pltpu.VMEM((_HP, D), jnp.float32),
            pltpu.VMEM((bpw,), jnp.float32),
            pltpu.SemaphoreType.DMA,
            pltpu.SemaphoreType.DMA,
        ],
    )
    def transe_kernel(lhs_hbm, rel_hbm, rhs_hbm, ent_hbm, relm_hbm, out_hbm,
                      li, ri, hi, lv, rv, hv, ov, sem_g, sem_o):
        wid = lax.axis_index("s") * info.num_cores + lax.axis_index("c")
        b0 = wid * bpw
        pltpu.sync_copy(lhs_hbm.at[pl.ds(b0, bpw)], li)
        pltpu.sync_copy(rel_hbm.at[pl.ds(b0, bpw)], ri)
        pltpu.sync_copy(rhs_hbm.at[pl.ds(b0, bpw)], hi)
        lanes = lax.iota(jnp.int32, 16)

        def fire(p, c):
            for half in range(_CH // 16):
                base = p * _HP + c * _CH + half * 16
                v1 = li[pl.ds(base, 16)]
                v2 = ri[pl.ds(base, 16)]
                v3 = hi[pl.ds(base, 16)]
                for i in range(16):
                    e = jnp.sum(jnp.where(lanes == i, v1, 0))
                    r = jnp.sum(jnp.where(lanes == i, v2, 0))
                    h = jnp.sum(jnp.where(lanes == i, v3, 0))
                    s = c * _CH + half * 16 + i
                    pltpu.async_copy(ent_hbm.at[e], lv.at[s], sem_g)
                    pltpu.async_copy(relm_hbm.at[r], rv.at[s], sem_g)
                    pltpu.async_copy(ent_hbm.at[h], hv.at[s], sem_g)

        def drain_chunk():
            pltpu.make_async_copy(
                ent_hbm.at[pl.ds(0, 3 * _CH)], lv.at[pl.ds(0, 3 * _CH)],
                sem_g).wait()

        lanes = lax.iota(jnp.int32, 16)

        def rsqrt(n):
            y = lax.bitcast_convert_type(n, jnp.int32)
            x = lax.bitcast_convert_type(0x5F3759DF - (y >> 1), jnp.float32)
            for _ in range(3):
                x = x * (1.5 - 0.5 * n * x * x)
            return x

        def sqrt(n):
            return n * rsqrt(n)

        def group(p, g):
            acc = jnp.zeros((16,), jnp.float32)

            def norm_scale(v0, v1):
                n = jnp.sum(v0 * v0 + v1 * v1)
                return jnp.minimum(1.0, rsqrt(n))

            for i in range(16):
                s = g * 16 + i
                l0 = lv[s, pl.ds(0, 16)]
                l1 = lv[s, pl.ds(16, 16)]
                r0 = rv[s, pl.ds(0, 16)]
                r1 = rv[s, pl.ds(16, 16)]
                h0 = hv[s, pl.ds(0, 16)]
                h1 = hv[s, pl.ds(16, 16)]
                sl = norm_scale(l0, l1)
                sr = norm_scale(r0, r1)
                sh = norm_scale(h0, h1)
                e0 = sl * l0 + sr * r0 - sh * h0
                e1 = sl * l1 + sr * r1 - sh * h1
                res = sqrt(jnp.sum(e0 * e0 + e1 * e1))
                acc = jnp.where(lanes == i, res, acc)
            ov[pl.ds(p * _HP + g * 16, 16)] = acc

        def run_pass(p, _):
            def fire_drain(c, _):
                fire(p, c)
                drain_chunk()
                return 0

            fire(p, 0)
            lax.fori_loop(1, nchp, fire_drain, 0)
            drain_chunk()

            def comp(g, _):
                group(p, g)
                return 0

            lax.fori_loop(0, _HP // 16, comp, 0)
            return 0

        lax.fori_loop(0, npass, run_pass, 0)
        pltpu.async_copy(ov, out_hbm.at[pl.ds(b0, bpw)], sem_o)
        pltpu.make_async_copy(ov, out_hbm.at[pl.ds(b0, bpw)], sem_o).wait()

    return transe_kernel(lhs, rel, rhs, ent_embeds, rel_embeds)


def kernel(triplets, ent_embeds, rel_embeds):
    B = triplets.shape[0]
    lhs = triplets[:, 0]
    rel = triplets[:, 1]
    rhs = triplets[:, 2]
    return _sc_transe(lhs, rel, rhs, ent_embeds, rel_embeds, B)
